# speedup vs baseline: 16.4318x; 16.4318x over previous
import functools

import jax
import jax.numpy as jnp
from jax import lax
from jax.experimental import pallas as pl
from jax.experimental.pallas import tpu as pltpu
from jax.experimental.pallas import tpu_sc as plsc

NN = 10000
NP = 10240
EE = 320000
EP = 327680
EROWS = EP // 128
TROWS = EROWS // 16
DI = 128
DH = 256
DO = 128

_MESH = plsc.VectorSubcoreMesh(core_axis_name="c", subcore_axis_name="s")
_NPT = NP // 16


def _deg_body(dst2d, deg_out, ones_v, idx_v, deg_sh):
    c = lax.axis_index("c")
    s = lax.axis_index("s")

    @pl.when(c == 0)
    def _():
        @pl.loop(0, _NPT // 16)
        def _fill(i):
            ones_v[pl.ds(i * 16, 16)] = jnp.full((16,), 1.0, jnp.float32)

        pltpu.sync_copy(ones_v, deg_sh.at[pl.ds(s * _NPT, _NPT)])
        plsc.subcore_barrier()

        @pl.loop(0, TROWS // 16)
        def _chunk(j):
            base = s * TROWS + j * 16
            pltpu.sync_copy(dst2d.at[pl.ds(base, 16)], idx_v)

            @pl.loop(0, 16)
            def _row(r):
                pltpu.sync_copy(ones_v.at[pl.ds(0, 128)],
                                deg_sh.at[idx_v.at[r]], add=True)

        plsc.subcore_barrier()
        pltpu.sync_copy(deg_sh.at[pl.ds(s * _NPT, _NPT)],
                        deg_out.at[pl.ds(s * _NPT, _NPT)])


_deg_call = functools.partial(
    pl.kernel,
    out_type=jax.ShapeDtypeStruct((NP,), jnp.float32),
    mesh=_MESH,
    scratch_types=[
        pltpu.VMEM((_NPT,), jnp.float32),
        pltpu.VMEM((16, 128), jnp.int32),
        pltpu.VMEM_SHARED((NP,), jnp.float32),
    ],
)(_deg_body)


def _prop_body(src2d, dst2d, tab_lo, tab_hi, out_lo, out_hi,
               srcb, dstb, rows, acc_sh, sem):
    c = lax.axis_index("c")
    s = lax.axis_index("s")

    def run(table, out):
        pltpu.sync_copy(table.at[pl.ds(s * _NPT, _NPT)],
                        acc_sh.at[pl.ds(s * _NPT, _NPT)])
        plsc.subcore_barrier()

        @pl.loop(0, TROWS // 16)
        def _chunk(j):
            base = s * TROWS + j * 16
            pltpu.sync_copy(src2d.at[pl.ds(base, 16)], srcb)
            pltpu.sync_copy(dst2d.at[pl.ds(base, 16)], dstb)

            @pl.loop(0, 16)
            def _row(r):
                pltpu.async_copy(table.at[srcb.at[r]], rows, sem).wait()
                pltpu.sync_copy(rows, acc_sh.at[dstb.at[r]], add=True)

        plsc.subcore_barrier()
        pltpu.sync_copy(acc_sh.at[pl.ds(s * _NPT, _NPT)],
                        out.at[pl.ds(s * _NPT, _NPT)])

    @pl.when(c == 0)
    def _():
        run(tab_lo, out_lo)

    @pl.when(c == 1)
    def _():
        run(tab_hi, out_hi)


_prop_call = functools.partial(
    pl.kernel,
    out_type=[jax.ShapeDtypeStruct((NP, 128), jnp.float32),
              jax.ShapeDtypeStruct((NP, 128), jnp.float32)],
    mesh=_MESH,
    scratch_types=[
        pltpu.VMEM((16, 128), jnp.int32),
        pltpu.VMEM((16, 128), jnp.int32),
        pltpu.VMEM((128, 128), jnp.float32),
        pltpu.VMEM_SHARED((NP, 128), jnp.float32),
        pltpu.SemaphoreType.DMA,
    ],
)(_prop_body)


def _prep_body(deg_ref, x_ref, w1_ref, zlo_ref, zhi_ref):
    dis = lax.rsqrt(deg_ref[...])
    z = jnp.dot(x_ref[...], w1_ref[...], preferred_element_type=jnp.float32)
    zs = z * dis
    zlo_ref[...] = zs[:, :128]
    zhi_ref[...] = zs[:, 128:]


def _mid_body(deg_ref, alo_ref, ahi_ref, wt_ref, wb_ref, blo_ref, bhi_ref,
              zlo_ref, zhi_ref):
    dis = lax.rsqrt(deg_ref[...])
    h_lo = jax.nn.relu(alo_ref[...] * dis + blo_ref[...])
    h_hi = jax.nn.relu(ahi_ref[...] * dis + bhi_ref[...])
    z2 = (jnp.dot(h_lo, wt_ref[...], preferred_element_type=jnp.float32)
          + jnp.dot(h_hi, wb_ref[...], preferred_element_type=jnp.float32))
    zs2 = z2 * dis
    zlo_ref[...] = zs2[:, :128]
    zhi_ref[...] = zs2[:, 128:]


def _final_body(deg_ref, alo_ref, ahi_ref, bmu_ref, bls_ref,
                mu_ref, ls_ref):
    dis = lax.rsqrt(deg_ref[...])
    mu_ref[...] = alo_ref[...] * dis + bmu_ref[...]
    ls_ref[...] = ahi_ref[...] * dis + bls_ref[...]


_BLK = 1024
_GRID = NP // _BLK

_row_spec = pl.BlockSpec((_BLK, 128), lambda i: (i, 0))
_deg_spec = pl.BlockSpec((_BLK, 1), lambda i: (i, 0))
_bias_spec = pl.BlockSpec((1, 128), lambda i: (0, 0))


def _prep_call(deg2, x_pad, w1):
    return pl.pallas_call(
        _prep_body,
        grid=(_GRID,),
        in_specs=[_deg_spec, _row_spec,
                  pl.BlockSpec((DI, DH), lambda i: (0, 0))],
        out_specs=[_row_spec, _row_spec],
        out_shape=[jax.ShapeDtypeStruct((NP, 128), jnp.float32)] * 2,
    )(deg2, x_pad, w1)


def _mid_call(deg2, alo, ahi, wt, wb, blo, bhi):
    return pl.pallas_call(
        _mid_body,
        grid=(_GRID,),
        in_specs=[_deg_spec, _row_spec, _row_spec,
                  pl.BlockSpec((128, DH), lambda i: (0, 0)),
                  pl.BlockSpec((128, DH), lambda i: (0, 0)),
                  _bias_spec, _bias_spec],
        out_specs=[_row_spec, _row_spec],
        out_shape=[jax.ShapeDtypeStruct((NP, 128), jnp.float32)] * 2,
    )(deg2, alo, ahi, wt, wb, blo, bhi)


def _final_call(deg2, alo, ahi, bmu, bls):
    return pl.pallas_call(
        _final_body,
        grid=(_GRID,),
        in_specs=[_deg_spec, _row_spec, _row_spec, _bias_spec, _bias_spec],
        out_specs=[_row_spec, _row_spec],
        out_shape=[jax.ShapeDtypeStruct((NP, 128), jnp.float32)] * 2,
    )(deg2, alo, ahi, bmu, bls)


def kernel(x, edge_index, W1, b1, W_mu, b_mu, W_ls, b_ls):
    src = edge_index[0]
    dst = edge_index[1]

    npad = EP - EE
    pad_src = (jnp.arange(npad, dtype=jnp.int32) * 61) % NN
    pad_dst = NN + (jnp.arange(npad, dtype=jnp.int32) % (NP - NN))
    src2d = jnp.concatenate([src, pad_src]).reshape(EROWS, 128)
    dst2d = jnp.concatenate([dst, pad_dst]).reshape(EROWS, 128)

    x_pad = jnp.pad(x, ((0, NP - NN), (0, 0)))

    wt = jnp.concatenate([W_mu[:128], W_ls[:128]], axis=1)
    wb = jnp.concatenate([W_mu[128:], W_ls[128:]], axis=1)
    blo = b1[:128].reshape(1, 128)
    bhi = b1[128:].reshape(1, 128)
    bmu = b_mu.reshape(1, 128)
    bls = b_ls.reshape(1, 128)

    deg = _deg_call(dst2d)
    deg2 = deg.reshape(NP, 1)

    zs_lo, zs_hi = _prep_call(deg2, x_pad, W1)
    acc_lo, acc_hi = _prop_call(src2d, dst2d, zs_lo, zs_hi)
    zs2_lo, zs2_hi = _mid_call(deg2, acc_lo, acc_hi, wt, wb, blo, bhi)
    acc2_lo, acc2_hi = _prop_call(src2d, dst2d, zs2_lo, zs2_hi)
    mu_p, ls_p = _final_call(deg2, acc2_lo, acc2_hi, bmu, bls)

    return (mu_p[:NN], ls_p[:NN])


# speedup vs baseline: 23.8104x; 1.4490x over previous
import functools

import jax
import jax.numpy as jnp
from jax import lax
from jax.experimental import pallas as pl
from jax.experimental.pallas import tpu as pltpu
from jax.experimental.pallas import tpu_sc as plsc

NN = 10000
NP = 10240
EE = 320000
EP = 327680
EROWS = EP // 128
TROWS = EROWS // 16
DI = 128
DH = 256
DO = 128

_MESH = plsc.VectorSubcoreMesh(core_axis_name="c", subcore_axis_name="s")
_NPT = NP // 16


def _deg_body(dst2d, deg_out, ones_v, idx_v, deg_sh):
    c = lax.axis_index("c")
    s = lax.axis_index("s")

    @pl.when(c == 0)
    def _():
        @pl.loop(0, _NPT // 16)
        def _fill(i):
            ones_v[pl.ds(i * 16, 16)] = jnp.full((16,), 1.0, jnp.float32)

        pltpu.sync_copy(ones_v, deg_sh.at[pl.ds(s * _NPT, _NPT)])
        plsc.subcore_barrier()

        @pl.loop(0, TROWS // 16)
        def _chunk(j):
            base = s * TROWS + j * 16
            pltpu.sync_copy(dst2d.at[pl.ds(base, 16)], idx_v)

            @pl.loop(0, 16)
            def _row(r):
                pltpu.sync_copy(ones_v.at[pl.ds(0, 128)],
                                deg_sh.at[idx_v.at[r]], add=True)

        plsc.subcore_barrier()
        pltpu.sync_copy(deg_sh.at[pl.ds(s * _NPT, _NPT)],
                        deg_out.at[pl.ds(s * _NPT, _NPT)])


_deg_call = functools.partial(
    pl.kernel,
    out_type=jax.ShapeDtypeStruct((NP,), jnp.float32),
    mesh=_MESH,
    scratch_types=[
        pltpu.VMEM((_NPT,), jnp.float32),
        pltpu.VMEM((16, 128), jnp.int32),
        pltpu.VMEM_SHARED((NP,), jnp.float32),
    ],
)(_deg_body)


def _prop_body(src2d, dst2d, tab_lo, tab_hi, out_lo, out_hi,
               srcb, dstb, buf0, buf1, acc_sh, sem0, sem1):
    c = lax.axis_index("c")
    s = lax.axis_index("s")

    def run(table, out):
        pltpu.sync_copy(table.at[pl.ds(s * _NPT, _NPT)],
                        acc_sh.at[pl.ds(s * _NPT, _NPT)])
        plsc.subcore_barrier()

        @pl.loop(0, TROWS // 16)
        def _chunk(j):
            base = s * TROWS + j * 16
            pltpu.sync_copy(src2d.at[pl.ds(base, 16)], srcb)
            pltpu.sync_copy(dst2d.at[pl.ds(base, 16)], dstb)
            pltpu.async_copy(table.at[srcb.at[0]], buf0, sem0)
            pltpu.async_copy(table.at[srcb.at[1]], buf1, sem1)

            @pl.loop(0, 16, step=2)
            def _row(r):
                for k, (buf, sem) in enumerate(((buf0, sem0), (buf1, sem1))):
                    idx = r + k
                    pltpu.make_async_copy(table.at[srcb.at[idx]],
                                          buf, sem).wait()
                    pltpu.sync_copy(buf, acc_sh.at[dstb.at[idx]], add=True)

                    @pl.when(idx + 2 < 16)
                    def _():
                        pltpu.async_copy(table.at[srcb.at[idx + 2]], buf, sem)

        plsc.subcore_barrier()
        pltpu.sync_copy(acc_sh.at[pl.ds(s * _NPT, _NPT)],
                        out.at[pl.ds(s * _NPT, _NPT)])

    @pl.when(c == 0)
    def _():
        run(tab_lo, out_lo)

    @pl.when(c == 1)
    def _():
        run(tab_hi, out_hi)


_prop_call = functools.partial(
    pl.kernel,
    out_type=[jax.ShapeDtypeStruct((NP, 128), jnp.float32),
              jax.ShapeDtypeStruct((NP, 128), jnp.float32)],
    mesh=_MESH,
    scratch_types=[
        pltpu.VMEM((16, 128), jnp.int32),
        pltpu.VMEM((16, 128), jnp.int32),
        pltpu.VMEM((128, 128), jnp.float32),
        pltpu.VMEM((128, 128), jnp.float32),
        pltpu.VMEM_SHARED((NP, 128), jnp.float32),
        pltpu.SemaphoreType.DMA,
        pltpu.SemaphoreType.DMA,
    ],
)(_prop_body)


def _prep_body(deg_ref, x_ref, w1_ref, zlo_ref, zhi_ref):
    dis = lax.rsqrt(deg_ref[...])
    z = jnp.dot(x_ref[...], w1_ref[...], preferred_element_type=jnp.float32)
    zs = z * dis
    zlo_ref[...] = zs[:, :128]
    zhi_ref[...] = zs[:, 128:]


def _mid_body(deg_ref, alo_ref, ahi_ref, wt_ref, wb_ref, blo_ref, bhi_ref,
              zlo_ref, zhi_ref):
    dis = lax.rsqrt(deg_ref[...])
    h_lo = jax.nn.relu(alo_ref[...] * dis + blo_ref[...])
    h_hi = jax.nn.relu(ahi_ref[...] * dis + bhi_ref[...])
    z2 = (jnp.dot(h_lo, wt_ref[...], preferred_element_type=jnp.float32)
          + jnp.dot(h_hi, wb_ref[...], preferred_element_type=jnp.float32))
    zs2 = z2 * dis
    zlo_ref[...] = zs2[:, :128]
    zhi_ref[...] = zs2[:, 128:]


def _final_body(deg_ref, alo_ref, ahi_ref, bmu_ref, bls_ref,
                mu_ref, ls_ref):
    dis = lax.rsqrt(deg_ref[...])
    mu_ref[...] = alo_ref[...] * dis + bmu_ref[...]
    ls_ref[...] = ahi_ref[...] * dis + bls_ref[...]


_BLK = 1024
_GRID = NP // _BLK

_row_spec = pl.BlockSpec((_BLK, 128), lambda i: (i, 0))
_deg_spec = pl.BlockSpec((_BLK, 1), lambda i: (i, 0))
_bias_spec = pl.BlockSpec((1, 128), lambda i: (0, 0))


def _prep_call(deg2, x_pad, w1):
    return pl.pallas_call(
        _prep_body,
        grid=(_GRID,),
        in_specs=[_deg_spec, _row_spec,
                  pl.BlockSpec((DI, DH), lambda i: (0, 0))],
        out_specs=[_row_spec, _row_spec],
        out_shape=[jax.ShapeDtypeStruct((NP, 128), jnp.float32)] * 2,
    )(deg2, x_pad, w1)


def _mid_call(deg2, alo, ahi, wt, wb, blo, bhi):
    return pl.pallas_call(
        _mid_body,
        grid=(_GRID,),
        in_specs=[_deg_spec, _row_spec, _row_spec,
                  pl.BlockSpec((128, DH), lambda i: (0, 0)),
                  pl.BlockSpec((128, DH), lambda i: (0, 0)),
                  _bias_spec, _bias_spec],
        out_specs=[_row_spec, _row_spec],
        out_shape=[jax.ShapeDtypeStruct((NP, 128), jnp.float32)] * 2,
    )(deg2, alo, ahi, wt, wb, blo, bhi)


def _final_call(deg2, alo, ahi, bmu, bls):
    return pl.pallas_call(
        _final_body,
        grid=(_GRID,),
        in_specs=[_deg_spec, _row_spec, _row_spec, _bias_spec, _bias_spec],
        out_specs=[_row_spec, _row_spec],
        out_shape=[jax.ShapeDtypeStruct((NP, 128), jnp.float32)] * 2,
    )(deg2, alo, ahi, bmu, bls)


def kernel(x, edge_index, W1, b1, W_mu, b_mu, W_ls, b_ls):
    src = edge_index[0]
    dst = edge_index[1]

    npad = EP - EE
    pad_src = (jnp.arange(npad, dtype=jnp.int32) * 61) % NN
    pad_dst = NN + (jnp.arange(npad, dtype=jnp.int32) % (NP - NN))
    src2d = jnp.concatenate([src, pad_src]).reshape(EROWS, 128)
    dst2d = jnp.concatenate([dst, pad_dst]).reshape(EROWS, 128)

    x_pad = jnp.pad(x, ((0, NP - NN), (0, 0)))

    wt = jnp.concatenate([W_mu[:128], W_ls[:128]], axis=1)
    wb = jnp.concatenate([W_mu[128:], W_ls[128:]], axis=1)
    blo = b1[:128].reshape(1, 128)
    bhi = b1[128:].reshape(1, 128)
    bmu = b_mu.reshape(1, 128)
    bls = b_ls.reshape(1, 128)

    deg = _deg_call(dst2d)
    deg2 = deg.reshape(NP, 1)

    zs_lo, zs_hi = _prep_call(deg2, x_pad, W1)
    acc_lo, acc_hi = _prop_call(src2d, dst2d, zs_lo, zs_hi)
    zs2_lo, zs2_hi = _mid_call(deg2, acc_lo, acc_hi, wt, wb, blo, bhi)
    acc2_lo, acc2_hi = _prop_call(src2d, dst2d, zs2_lo, zs2_hi)
    mu_p, ls_p = _final_call(deg2, acc2_lo, acc2_hi, bmu, bls)

    return (mu_p[:NN], ls_p[:NN])
